# baseline (speedup 1.0000x reference)
import jax
import jax.numpy as jnp
from jax import lax
from jax.experimental import pallas as pl
from jax.experimental.pallas import tpu as pltpu
from jax.experimental.pallas import tpu_sc as plsc

_N, _T, _C = 4096, 200, 64
_ALPHA = 0.5
_L = 16
_NC, _NS = 2, 16
_NW = _NC * _NS
_EPW = _N * _T // _NW
_K = 128
_NCHUNK = _EPW // _K
_VPC = _K // _L
_NBUF = 10
assert _NCHUNK % _NBUF == 0


def _sc_body(logp_hbm, z_hbm, y_hbm, out_hbm, z_v, y_v, *rest):
    idxs = rest[:_NBUF]
    gbufs = rest[_NBUF:2 * _NBUF]
    acc_v = rest[2 * _NBUF]
    sems = rest[2 * _NBUF + 1:]

    wid = lax.axis_index("s") * _NC + lax.axis_index("c")

    zy_copies = []
    for t1 in range(_T // 8):
        off = (t1 * _NW + wid) * 1024
        dst = pl.ds(t1 * 1024, 1024)
        for src_hbm, dst_v in ((z_hbm, z_v), (y_hbm, y_v)):
            c = pltpu.make_async_copy(src_hbm.at[pl.ds(off, 1024)],
                                      dst_v.at[dst], sems[0])
            c.start()
            zy_copies.append(c)
    for c in zy_copies:
        c.wait()

    def zy(j, v):
        o = j * _K + v * _L
        return z_v[pl.ds(o, _L)], y_v[pl.ds(o, _L)]

    def fire(j, b):
        for v in range(_VPC):
            z, y = zy(j, v)
            tt = jnp.minimum(j + z, _T - 1)
            flat = (tt * (_C * _N)
                    + lax.shift_right_logical(y, 3) * (8 * _N)
                    + wid * 1024
                    + jnp.bitwise_and(y, 7) * _K
                    + v * _L + lax.iota(jnp.int32, _L))
            idxs[b][pl.ds(v * _L, _L)] = flat
        pltpu.make_async_copy(logp_hbm.at[idxs[b]], gbufs[b],
                              sems[b]).start()

    def consume(j, b, acc):
        pltpu.make_async_copy(logp_hbm.at[idxs[b]], gbufs[b],
                              sems[b]).wait()
        sa = 0.5 * (1.0 - j.astype(jnp.float32) * (1.0 / _T))
        sb = sa * (1.0 / _T)
        for v in range(_VPC):
            z, _ = zy(j, v)
            lc = gbufs[b][pl.ds(v * _L, _L)]
            w = sa - sb * z.astype(jnp.float32)
            acc = acc + (-_ALPHA) * lc - jnp.exp(lc) * w
        return acc

    for b in range(_NBUF):
        fire(jnp.int32(b), b)

    def body(jb, acc):
        j = jb * _NBUF
        for b in range(_NBUF):
            acc = consume(j + b, b, acc)
            fire(j + b + _NBUF, b)
        return acc

    acc = jnp.zeros((_L,), jnp.float32)
    acc = lax.fori_loop(0, _NCHUNK // _NBUF - 1, body, acc)
    base = jnp.int32(_NCHUNK - _NBUF)
    for b in range(_NBUF):
        acc = consume(base + b, b, acc)

    acc_v[...] = acc
    pltpu.sync_copy(acc_v, out_hbm.at[wid])


_sc_kernel = pl.kernel(
    _sc_body,
    mesh=plsc.VectorSubcoreMesh(core_axis_name="c", subcore_axis_name="s"),
    out_type=jax.ShapeDtypeStruct((_NW, _L), jnp.float32),
    scratch_types=(
        [pltpu.VMEM((_EPW,), jnp.int32)] * 2
        + [pltpu.VMEM((_K,), jnp.int32)] * _NBUF
        + [pltpu.VMEM((_K,), jnp.float32)] * _NBUF
        + [pltpu.VMEM((_L,), jnp.float32)]
        + [pltpu.SemaphoreType.DMA] * _NBUF
    ),
)


def _tile_flat(a):
    return (a.T.reshape(_T // 8, 8, _N // 128, 128)
            .transpose(0, 2, 1, 3)
            .reshape(-1))


def kernel(log_class_probabilities, timestamps_left, y_true):
    logp_flat = (log_class_probabilities
                 .transpose(1, 2, 0)
                 .reshape(_T, _C // 8, 8, _N // 128, 128)
                 .transpose(0, 1, 3, 2, 4)
                 .reshape(-1))
    z = _tile_flat(timestamps_left.astype(jnp.int32))
    y = _tile_flat(y_true.astype(jnp.int32))
    partials = _sc_kernel(logp_flat, z, y)
    return partials.sum() * (1.0 / _N)


# speedup vs baseline: 1.0092x; 1.0092x over previous
import jax
import jax.numpy as jnp
from jax import lax
from jax.experimental import pallas as pl
from jax.experimental.pallas import tpu as pltpu
from jax.experimental.pallas import tpu_sc as plsc

_N, _T, _C = 4096, 200, 64
_ALPHA = 0.5
_L = 16
_NC, _NS = 2, 16
_NW = _NC * _NS
_EPW = _N * _T // _NW
_K = 128
_NCHUNK = _EPW // _K
_VPC = _K // _L
_NBUF = 8
assert _NCHUNK % _NBUF == 0


def _sc_body(logp_hbm, z_hbm, y_hbm, out_hbm, z_v, y_v, *rest):
    idxs = rest[:_NBUF]
    gbufs = rest[_NBUF:2 * _NBUF]
    acc_v = rest[2 * _NBUF]
    sems = rest[2 * _NBUF + 1:2 * _NBUF + 1 + _NBUF]
    sem_stage = rest[2 * _NBUF + 1 + _NBUF]

    wid = lax.axis_index("s") * _NC + lax.axis_index("c")

    def stage_block(t1):
        off = (t1 * _NW + wid) * 1024
        dst = pl.ds(t1 * 1024, 1024)
        cs = []
        for src_hbm, dst_v in ((z_hbm, z_v), (y_hbm, y_v)):
            c = pltpu.make_async_copy(src_hbm.at[pl.ds(off, 1024)],
                                      dst_v.at[dst], sem_stage)
            c.start()
            cs.append(c)
        return cs

    for c in stage_block(0):
        c.wait()

    def zy(j, v):
        o = j * _K + v * _L
        return z_v[pl.ds(o, _L)], y_v[pl.ds(o, _L)]

    def fire(j, b):
        for v in range(_VPC):
            z, y = zy(j, v)
            tt = jnp.minimum(j + z, _T - 1)
            flat = (tt * (_C * _N)
                    + lax.shift_right_logical(y, 3) * (8 * _N)
                    + wid * 1024
                    + jnp.bitwise_and(y, 7) * _K
                    + v * _L + lax.iota(jnp.int32, _L))
            idxs[b][pl.ds(v * _L, _L)] = flat
        pltpu.make_async_copy(logp_hbm.at[idxs[b]], gbufs[b],
                              sems[b]).start()

    def consume(j, b, acc):
        pltpu.make_async_copy(logp_hbm.at[idxs[b]], gbufs[b],
                              sems[b]).wait()
        sa = 0.5 * (1.0 - j.astype(jnp.float32) * (1.0 / _T))
        sb = sa * (1.0 / _T)
        for v in range(_VPC):
            z, _ = zy(j, v)
            lc = gbufs[b][pl.ds(v * _L, _L)]
            w = sa - sb * z.astype(jnp.float32)
            acc = acc + (-_ALPHA) * lc - jnp.exp(lc) * w
        return acc

    for b in range(_NBUF):
        fire(jnp.int32(b), b)

    zy_copies = []
    for t1 in range(1, _T // 8):
        zy_copies.extend(stage_block(t1))
    for c in zy_copies:
        c.wait()

    def body(jb, acc):
        j = jb * _NBUF
        for b in range(_NBUF):
            acc = consume(j + b, b, acc)
            fire(j + b + _NBUF, b)
        return acc

    acc = jnp.zeros((_L,), jnp.float32)
    acc = lax.fori_loop(0, _NCHUNK // _NBUF - 1, body, acc)
    base = jnp.int32(_NCHUNK - _NBUF)
    for b in range(_NBUF):
        acc = consume(base + b, b, acc)

    acc_v[...] = acc
    pltpu.sync_copy(acc_v, out_hbm.at[wid])


_sc_kernel = pl.kernel(
    _sc_body,
    mesh=plsc.VectorSubcoreMesh(core_axis_name="c", subcore_axis_name="s"),
    out_type=jax.ShapeDtypeStruct((_NW, _L), jnp.float32),
    scratch_types=(
        [pltpu.VMEM((_EPW,), jnp.int32)] * 2
        + [pltpu.VMEM((_K,), jnp.int32)] * _NBUF
        + [pltpu.VMEM((_K,), jnp.float32)] * _NBUF
        + [pltpu.VMEM((_L,), jnp.float32)]
        + [pltpu.SemaphoreType.DMA] * (_NBUF + 1)
    ),
)


def _tile_flat(a):
    return (a.T.reshape(_T // 8, 8, _N // 128, 128)
            .transpose(0, 2, 1, 3)
            .reshape(-1))


def kernel(log_class_probabilities, timestamps_left, y_true):
    logp_flat = (log_class_probabilities
                 .transpose(1, 2, 0)
                 .reshape(_T, _C // 8, 8, _N // 128, 128)
                 .transpose(0, 1, 3, 2, 4)
                 .reshape(-1))
    z = _tile_flat(timestamps_left.astype(jnp.int32))
    y = _tile_flat(y_true.astype(jnp.int32))
    partials = _sc_kernel(logp_flat, z, y)
    return partials.sum() * (1.0 / _N)
